# baseline (speedup 1.0000x reference)
import jax
import jax.numpy as jnp
from jax.experimental import pallas as pl

N_NODES = 54
N_PAD = 64
E_EDGES = 2862
D_EMB = 256
EMB_FLAT = N_NODES * D_EMB

_OFF_B1, _OFF_B2, _OFF_B3 = 0, 512, 1024
_OFF_BO1, _OFF_WO2, _OFF_BO2 = 1280, 1408, 1536
_OFF_BG1, _OFF_BG2, _OFF_BG3, _OFF_G = 1664, 1792, 1920, 1984
_ROW_LEN = 2048


def _net_body(ei_ref, feat_ref, w1_ref, w2_ref, w3_ref, wgp_ref, wo1_ref,
              row_ref, out_ref):
    ei = ei_ref[...]
    src = jnp.broadcast_to(ei[0], (2 * N_PAD, E_EDGES))
    dst = jnp.broadcast_to(ei[1], (2 * N_PAD, E_EDGES))
    sub = jax.lax.broadcasted_iota(jnp.int32, (2 * N_PAD, E_EDGES), 0)
    st = (jnp.where(sub < N_PAD, src, dst) == (sub & (N_PAD - 1))
          ).astype(jnp.float32)
    m = jax.lax.dot_general(st, st, (((1,), (1,)), ((), ())),
                            preferred_element_type=jnp.float32)
    a = m[N_PAD:, :N_PAD]
    rin = jax.lax.rsqrt(jnp.maximum(jnp.sum(a, axis=1, keepdims=True), 1.0))
    rout = jax.lax.rsqrt(jnp.maximum(jnp.sum(a, axis=0, keepdims=True), 1.0))
    a_norm = a * rin * rout

    row = row_ref[...]
    b1 = row[:, _OFF_B1:_OFF_B1 + 512]
    b2 = row[:, _OFF_B2:_OFF_B2 + 512]
    b3 = row[:, _OFF_B3:_OFF_B3 + 256]

    x = jnp.dot(feat_ref[...], w1_ref[...], preferred_element_type=jnp.float32)
    h = jnp.maximum(jnp.dot(a_norm[:, :N_NODES], x,
                            preferred_element_type=jnp.float32) + b1, 0.0)
    x = jnp.dot(h, w2_ref[...], preferred_element_type=jnp.float32)
    h = jnp.maximum(jnp.dot(a_norm, x,
                            preferred_element_type=jnp.float32) + b2, 0.0)
    x = jnp.dot(h, w3_ref[...], preferred_element_type=jnp.float32)
    h = jnp.maximum(jnp.dot(a_norm, x,
                            preferred_element_type=jnp.float32) + b3, 0.0)
    rowi = jax.lax.broadcasted_iota(jnp.int32, (N_PAD, D_EMB), 0)
    emb = jnp.where(rowi < N_NODES, h, 0.0)

    g = row[:, _OFF_G:_OFF_G + 64]
    g = jnp.maximum(jnp.dot(g, wgp_ref[:, 0:16],
                            preferred_element_type=jnp.float32)
                    + row[:, _OFF_BG1:_OFF_BG1 + 16], 0.0)
    g = jnp.maximum(jnp.dot(g, wgp_ref[0:16, 16:32],
                            preferred_element_type=jnp.float32)
                    + row[:, _OFF_BG2:_OFF_BG2 + 16], 0.0)
    g = jnp.maximum(jnp.dot(g, wgp_ref[0:16, 32:96],
                            preferred_element_type=jnp.float32)
                    + row[:, _OFF_BG3:_OFF_BG3 + 64], 0.0)

    flat = emb.reshape(1, N_PAD * D_EMB)
    o = (jnp.dot(flat[:, :EMB_FLAT], wo1_ref[:EMB_FLAT, :],
                 preferred_element_type=jnp.float32)
         + jnp.dot(g, wo1_ref[EMB_FLAT:, :],
                   preferred_element_type=jnp.float32)
         + row[:, _OFF_BO1:_OFF_BO1 + 85])
    o = jnp.maximum(o, 0.0)
    o = (jnp.sum(o * row[:, _OFF_WO2:_OFF_WO2 + 85], axis=1, keepdims=True)
         + row[:, _OFF_BO2:_OFF_BO2 + 1])
    out_ref[...] = jax.nn.sigmoid(o)


def _pad_to(x, n):
    return jnp.pad(x, (0, n - x.shape[0]))


def kernel(feat, edge_index, globalFeats, isTrain, W1, b1, W2, b2, W3, b3,
           Wg1, bg1, Wg2, bg2, Wg3, bg3, Wo1, bo1, Wo2, bo2):
    del isTrain
    row = jnp.concatenate([
        b1, b2, b3,
        _pad_to(bo1, 128), _pad_to(Wo2[:, 0], 128), _pad_to(bo2, 128),
        _pad_to(bg1, 128), _pad_to(bg2, 128), bg3, globalFeats,
    ]).reshape(1, _ROW_LEN)
    wgp = jnp.concatenate([
        Wg1, jnp.pad(Wg2, ((0, 48), (0, 0))), jnp.pad(Wg3, ((0, 48), (0, 0))),
    ], axis=1)
    out = pl.pallas_call(
        _net_body,
        out_shape=jax.ShapeDtypeStruct((1, 1), jnp.float32),
    )(edge_index.astype(jnp.int32).reshape(2, 1, E_EDGES),
      feat, W1, W2, W3, wgp, Wo1, row)
    return out.reshape(1)


# speedup vs baseline: 1.2821x; 1.2821x over previous
import jax
import jax.numpy as jnp
from jax.experimental import pallas as pl
from jax.experimental.pallas import tpu as pltpu

N_NODES = 54
N_PAD = 64
E_EDGES = 2862
D_IN, D_HID, D_EMB = 512, 512, 256
EMB_FLAT = N_NODES * D_EMB
WO1_ROWS = EMB_FLAT + 64
WO1_HALF = 6912


def _net_body(ei_ref, feat_ref, g_ref,
              w1_ref, b1_ref, w2_ref, b2_ref, w3_ref, b3_ref,
              wg1_ref, bg1_ref, wg2_ref, bg2_ref, wg3_ref, bg3_ref,
              wo1_ref, bo1_ref, wo2t_ref, bo2_ref, out_ref,
              w1_s, w2_s, w3_s, wo1a_s, wo1b_s,
              sem1, sem2, sem3, sem4, sem5):
    cp1 = pltpu.make_async_copy(w1_ref, w1_s, sem1)
    cp2 = pltpu.make_async_copy(w2_ref, w2_s, sem2)
    cp3 = pltpu.make_async_copy(w3_ref, w3_s, sem3)
    cp4 = pltpu.make_async_copy(wo1_ref.at[:WO1_HALF], wo1a_s, sem4)
    cp5 = pltpu.make_async_copy(wo1_ref.at[WO1_HALF:], wo1b_s, sem5)
    cp1.start(); cp2.start(); cp3.start(); cp4.start(); cp5.start()

    ei = ei_ref[...]
    src = jnp.broadcast_to(ei[0], (2 * N_PAD, E_EDGES))
    dst = jnp.broadcast_to(ei[1], (2 * N_PAD, E_EDGES))
    sub = jax.lax.broadcasted_iota(jnp.int32, (2 * N_PAD, E_EDGES), 0)
    st = (jnp.where(sub < N_PAD, src, dst) == (sub & (N_PAD - 1))
          ).astype(jnp.float32)
    m = jax.lax.dot_general(st, st, (((1,), (1,)), ((), ())),
                            preferred_element_type=jnp.float32)
    a = m[N_PAD:, :N_PAD]
    rin = jax.lax.rsqrt(jnp.maximum(jnp.sum(a, axis=1, keepdims=True), 1.0))
    rout = jax.lax.rsqrt(jnp.maximum(jnp.sum(a, axis=0, keepdims=True), 1.0))
    a_norm = a * rin * rout

    g = jnp.maximum(jnp.dot(g_ref[...], wg1_ref[...],
                            preferred_element_type=jnp.float32) + bg1_ref[...], 0.0)
    g = jnp.maximum(jnp.dot(g, wg2_ref[...],
                            preferred_element_type=jnp.float32) + bg2_ref[...], 0.0)
    g = jnp.maximum(jnp.dot(g, wg3_ref[...],
                            preferred_element_type=jnp.float32) + bg3_ref[...], 0.0)

    cp1.wait()
    x = jnp.dot(feat_ref[...], w1_s[...], preferred_element_type=jnp.float32)
    h = jnp.maximum(jnp.dot(a_norm[:, :N_NODES], x,
                            preferred_element_type=jnp.float32) + b1_ref[...], 0.0)
    cp2.wait()
    x = jnp.dot(h, w2_s[...], preferred_element_type=jnp.float32)
    h = jnp.maximum(jnp.dot(a_norm, x,
                            preferred_element_type=jnp.float32) + b2_ref[...], 0.0)
    cp3.wait()
    x = jnp.dot(h, w3_s[...], preferred_element_type=jnp.float32)
    h = jnp.maximum(jnp.dot(a_norm, x,
                            preferred_element_type=jnp.float32) + b3_ref[...], 0.0)
    rowi = jax.lax.broadcasted_iota(jnp.int32, (N_PAD, D_EMB), 0)
    emb = jnp.where(rowi < N_NODES, h, 0.0)

    flat = emb.reshape(1, N_PAD * D_EMB)
    cp4.wait()
    o = jnp.dot(flat[:, :WO1_HALF], wo1a_s[...],
                preferred_element_type=jnp.float32)
    cp5.wait()
    o = (o + jnp.dot(flat[:, WO1_HALF:EMB_FLAT], wo1b_s[:EMB_FLAT - WO1_HALF, :],
                     preferred_element_type=jnp.float32)
         + jnp.dot(g, wo1b_s[EMB_FLAT - WO1_HALF:, :],
                   preferred_element_type=jnp.float32)
         + bo1_ref[...])
    o = jnp.maximum(o, 0.0)
    o = jnp.sum(o * wo2t_ref[...], axis=1, keepdims=True) + bo2_ref[...]
    out_ref[...] = jax.nn.sigmoid(o)


def kernel(feat, edge_index, globalFeats, isTrain, W1, b1, W2, b2, W3, b3,
           Wg1, bg1, Wg2, bg2, Wg3, bg3, Wo1, bo1, Wo2, bo2):
    del isTrain
    vmem = pl.BlockSpec(memory_space=pltpu.VMEM)
    hbm = pl.BlockSpec(memory_space=pl.ANY)
    out = pl.pallas_call(
        _net_body,
        out_shape=jax.ShapeDtypeStruct((1, 1), jnp.float32),
        in_specs=[vmem, vmem, vmem,
                  hbm, vmem, hbm, vmem, hbm, vmem,
                  vmem, vmem, vmem, vmem, vmem, vmem,
                  hbm, vmem, vmem, vmem],
        scratch_shapes=[
            pltpu.VMEM((D_IN, D_HID), jnp.float32),
            pltpu.VMEM((D_HID, D_HID), jnp.float32),
            pltpu.VMEM((D_HID, D_EMB), jnp.float32),
            pltpu.VMEM((WO1_HALF, 85), jnp.float32),
            pltpu.VMEM((WO1_ROWS - WO1_HALF, 85), jnp.float32),
            pltpu.SemaphoreType.DMA, pltpu.SemaphoreType.DMA,
            pltpu.SemaphoreType.DMA, pltpu.SemaphoreType.DMA,
            pltpu.SemaphoreType.DMA,
        ],
    )(edge_index.astype(jnp.int32).reshape(2, 1, E_EDGES),
      feat, globalFeats.reshape(1, -1),
      W1, b1.reshape(1, -1), W2, b2.reshape(1, -1), W3, b3.reshape(1, -1),
      Wg1, bg1.reshape(1, -1), Wg2, bg2.reshape(1, -1), Wg3, bg3.reshape(1, -1),
      Wo1, bo1.reshape(1, -1), Wo2.reshape(1, -1), bo2.reshape(1, 1))
    return out.reshape(1)
